# Initial kernel scaffold; baseline (speedup 1.0000x reference)
import jax
import jax.numpy as jnp
from jax.experimental import pallas as pl


def kernel(x, edge_index, edge_weight, Wz, bz, Wr, br, Wh, bh, Wl, bl):
    raise NotImplementedError("write your pallas kernel here")


# speedup vs baseline: 1.2063x; 1.2063x over previous
import jax
import jax.numpy as jnp
from jax.experimental import pallas as pl
from jax.experimental.pallas import tpu as pltpu

_N = 10000
_F_IN = 128
_F_OUT = 32
_NUM_CLASSES = 10
_TILE = 1000


def _fused_gcn_cell(x_ref, wz_ref, bz_ref, wh_ref, bh_ref, wl_ref, bl_ref,
                    o_ref):
    x = x_ref[...]
    az = wz_ref[0, 0, :_F_IN, :] + wz_ref[1, 0, :_F_IN, :]
    ah = wh_ref[0, 0, :_F_IN, :] + wh_ref[1, 0, :_F_IN, :]
    z = jax.nn.sigmoid(
        jnp.dot(x, az, preferred_element_type=jnp.float32) + bz_ref[...])
    ht = jnp.tanh(
        jnp.dot(x, ah, preferred_element_type=jnp.float32) + bh_ref[...])
    h = jax.nn.relu((1.0 - z) * ht)
    o_ref[...] = (
        jnp.dot(h, wl_ref[...], preferred_element_type=jnp.float32)
        + bl_ref[...])


def kernel(x, edge_index, edge_weight, Wz, bz, Wr, br, Wh, bh, Wl, bl):
    del edge_index, edge_weight, Wr, br
    grid = _N // _TILE
    w_spec = pl.BlockSpec(Wz.shape, lambda i: (0, 0, 0, 0))
    b_spec = pl.BlockSpec((1, _F_OUT), lambda i: (0, 0))
    return pl.pallas_call(
        _fused_gcn_cell,
        grid=(grid,),
        in_specs=[
            pl.BlockSpec((_TILE, _F_IN), lambda i: (i, 0)),
            w_spec,
            b_spec,
            w_spec,
            b_spec,
            pl.BlockSpec((_F_OUT, _NUM_CLASSES), lambda i: (0, 0)),
            pl.BlockSpec((1, _NUM_CLASSES), lambda i: (0, 0)),
        ],
        out_specs=pl.BlockSpec((_TILE, _NUM_CLASSES), lambda i: (i, 0)),
        out_shape=jax.ShapeDtypeStruct((_N, _NUM_CLASSES), jnp.float32),
        compiler_params=pltpu.CompilerParams(
            dimension_semantics=("arbitrary",),
        ),
    )(x, Wz, bz.reshape(1, _F_OUT), Wh, bh.reshape(1, _F_OUT), Wl,
      bl.reshape(1, _NUM_CLASSES))


# speedup vs baseline: 1.4688x; 1.2176x over previous
import jax
import jax.numpy as jnp
from jax.experimental import pallas as pl
from jax.experimental.pallas import tpu as pltpu

_N = 10000
_F_IN = 128
_F_OUT = 32
_NUM_CLASSES = 10
_TILE = 2000


def _fused_gcn_cell(x_ref, wz_ref, bz_ref, wh_ref, bh_ref, wl_ref, bl_ref,
                    o_ref, comb_ref, bcat_ref):
    @pl.when(pl.program_id(0) == 0)
    def _prep():
        az = (wz_ref[0, 0, :_F_IN, :] + wz_ref[1, 0, :_F_IN, :]) * -0.5
        ah = wh_ref[0, 0, :_F_IN, :] + wh_ref[1, 0, :_F_IN, :]
        comb_ref[...] = jnp.concatenate([az, ah], axis=1)
        bcat_ref[...] = jnp.concatenate(
            [bz_ref[...] * -0.5, bh_ref[...]], axis=1)

    x = x_ref[...]
    g = jnp.dot(x, comb_ref[...], preferred_element_type=jnp.float32) \
        + bcat_ref[...]
    t = jnp.tanh(g)
    one_minus_z = 1.0 + t[:, :_F_OUT]
    ht = t[:, _F_OUT:]
    h = jax.nn.relu(one_minus_z * ht)
    o_ref[...] = (
        jnp.dot(h, wl_ref[...] * 0.5, preferred_element_type=jnp.float32)
        + bl_ref[...])


def kernel(x, edge_index, edge_weight, Wz, bz, Wr, br, Wh, bh, Wl, bl):
    del edge_index, edge_weight, Wr, br
    grid = _N // _TILE
    w_spec = pl.BlockSpec(Wz.shape, lambda i: (0, 0, 0, 0))
    b_spec = pl.BlockSpec((1, _F_OUT), lambda i: (0, 0))
    return pl.pallas_call(
        _fused_gcn_cell,
        grid=(grid,),
        in_specs=[
            pl.BlockSpec((_TILE, _F_IN), lambda i: (i, 0)),
            w_spec,
            b_spec,
            w_spec,
            b_spec,
            pl.BlockSpec((_F_OUT, _NUM_CLASSES), lambda i: (0, 0)),
            pl.BlockSpec((1, _NUM_CLASSES), lambda i: (0, 0)),
        ],
        out_specs=pl.BlockSpec((_TILE, _NUM_CLASSES), lambda i: (i, 0)),
        out_shape=jax.ShapeDtypeStruct((_N, _NUM_CLASSES), jnp.float32),
        scratch_shapes=[
            pltpu.VMEM((_F_IN, 2 * _F_OUT), jnp.float32),
            pltpu.VMEM((1, 2 * _F_OUT), jnp.float32),
        ],
        compiler_params=pltpu.CompilerParams(
            dimension_semantics=("arbitrary",),
        ),
    )(x, Wz, bz.reshape(1, _F_OUT), Wh, bh.reshape(1, _F_OUT), Wl,
      bl.reshape(1, _NUM_CLASSES))


# speedup vs baseline: 1.5654x; 1.0657x over previous
import jax
import jax.numpy as jnp
from jax.experimental import pallas as pl
from jax.experimental.pallas import tpu as pltpu

_N = 10000
_F_IN = 128
_F_OUT = 32
_NUM_CLASSES = 10
_TILE = 5000


def _fused_gcn_cell(x_ref, wz_ref, bz_ref, wh_ref, bh_ref, wl_ref, bl_ref,
                    o_ref, comb_ref, bcat_ref):
    @pl.when(pl.program_id(0) == 0)
    def _prep():
        az = (wz_ref[0, 0, :_F_IN, :] + wz_ref[1, 0, :_F_IN, :]) * -0.5
        ah = wh_ref[0, 0, :_F_IN, :] + wh_ref[1, 0, :_F_IN, :]
        comb_ref[...] = jnp.concatenate([az, ah], axis=1)
        bcat_ref[...] = jnp.concatenate(
            [bz_ref[...] * -0.5, bh_ref[...]], axis=1)

    x = x_ref[...]
    g = jnp.dot(x, comb_ref[...], preferred_element_type=jnp.float32) \
        + bcat_ref[...]
    t = jnp.tanh(g)
    one_minus_z = 1.0 + t[:, :_F_OUT]
    ht = t[:, _F_OUT:]
    h = jax.nn.relu(one_minus_z * ht)
    o_ref[...] = (
        jnp.dot(h, wl_ref[...] * 0.5, preferred_element_type=jnp.float32)
        + bl_ref[...])


def kernel(x, edge_index, edge_weight, Wz, bz, Wr, br, Wh, bh, Wl, bl):
    del edge_index, edge_weight, Wr, br
    grid = _N // _TILE
    w_spec = pl.BlockSpec(Wz.shape, lambda i: (0, 0, 0, 0))
    b_spec = pl.BlockSpec((1, _F_OUT), lambda i: (0, 0))
    return pl.pallas_call(
        _fused_gcn_cell,
        grid=(grid,),
        in_specs=[
            pl.BlockSpec((_TILE, _F_IN), lambda i: (i, 0)),
            w_spec,
            b_spec,
            w_spec,
            b_spec,
            pl.BlockSpec((_F_OUT, _NUM_CLASSES), lambda i: (0, 0)),
            pl.BlockSpec((1, _NUM_CLASSES), lambda i: (0, 0)),
        ],
        out_specs=pl.BlockSpec((_TILE, _NUM_CLASSES), lambda i: (i, 0)),
        out_shape=jax.ShapeDtypeStruct((_N, _NUM_CLASSES), jnp.float32),
        scratch_shapes=[
            pltpu.VMEM((_F_IN, 2 * _F_OUT), jnp.float32),
            pltpu.VMEM((1, 2 * _F_OUT), jnp.float32),
        ],
        compiler_params=pltpu.CompilerParams(
            dimension_semantics=("arbitrary",),
        ),
    )(x, Wz, bz.reshape(1, _F_OUT), Wh, bh.reshape(1, _F_OUT), Wl,
      bl.reshape(1, _NUM_CLASSES))


# speedup vs baseline: 1.5951x; 1.0190x over previous
import jax
import jax.numpy as jnp
from jax.experimental import pallas as pl
from jax.experimental.pallas import tpu as pltpu

_N = 10000
_F_IN = 128
_F_OUT = 32
_NUM_CLASSES = 10
_TILE = 10000


def _fused_gcn_cell(x_ref, wz_ref, bz_ref, wh_ref, bh_ref, wl_ref, bl_ref,
                    o_ref, comb_ref, bcat_ref):
    @pl.when(pl.program_id(0) == 0)
    def _prep():
        az = (wz_ref[0, 0, :_F_IN, :] + wz_ref[1, 0, :_F_IN, :]) * -0.5
        ah = wh_ref[0, 0, :_F_IN, :] + wh_ref[1, 0, :_F_IN, :]
        comb_ref[...] = jnp.concatenate([az, ah], axis=1)
        bcat_ref[...] = jnp.concatenate(
            [bz_ref[...] * -0.5, bh_ref[...]], axis=1)

    x = x_ref[...]
    g = jnp.dot(x, comb_ref[...], preferred_element_type=jnp.float32) \
        + bcat_ref[...]
    t = jnp.tanh(g)
    one_minus_z = 1.0 + t[:, :_F_OUT]
    ht = t[:, _F_OUT:]
    h = jax.nn.relu(one_minus_z * ht)
    o_ref[...] = (
        jnp.dot(h, wl_ref[...] * 0.5, preferred_element_type=jnp.float32)
        + bl_ref[...])


def kernel(x, edge_index, edge_weight, Wz, bz, Wr, br, Wh, bh, Wl, bl):
    del edge_index, edge_weight, Wr, br
    grid = _N // _TILE
    w_spec = pl.BlockSpec(Wz.shape, lambda i: (0, 0, 0, 0))
    b_spec = pl.BlockSpec((1, _F_OUT), lambda i: (0, 0))
    return pl.pallas_call(
        _fused_gcn_cell,
        grid=(grid,),
        in_specs=[
            pl.BlockSpec((_TILE, _F_IN), lambda i: (i, 0)),
            w_spec,
            b_spec,
            w_spec,
            b_spec,
            pl.BlockSpec((_F_OUT, _NUM_CLASSES), lambda i: (0, 0)),
            pl.BlockSpec((1, _NUM_CLASSES), lambda i: (0, 0)),
        ],
        out_specs=pl.BlockSpec((_TILE, _NUM_CLASSES), lambda i: (i, 0)),
        out_shape=jax.ShapeDtypeStruct((_N, _NUM_CLASSES), jnp.float32),
        scratch_shapes=[
            pltpu.VMEM((_F_IN, 2 * _F_OUT), jnp.float32),
            pltpu.VMEM((1, 2 * _F_OUT), jnp.float32),
        ],
        compiler_params=pltpu.CompilerParams(
            dimension_semantics=("arbitrary",),
        ),
    )(x, Wz, bz.reshape(1, _F_OUT), Wh, bh.reshape(1, _F_OUT), Wl,
      bl.reshape(1, _NUM_CLASSES))
